# baseline (speedup 1.0000x reference)
import functools

import jax
import jax.numpy as jnp
from jax import lax
from jax.experimental import pallas as pl
from jax.experimental.pallas import tpu as pltpu
from jax.experimental.pallas import tpu_sc as plsc

N, E, D = 10000, 320000, 128
NC, NS = 2, 16
NW = NC * NS
EPW = E // NW
RPW = D // NW
CHUNK = 6400
NCH = E // CHUNK
NB = 1000

_mesh = plsc.VectorSubcoreMesh(
    core_axis_name="c", subcore_axis_name="s", num_cores=NC, num_subcores=NS
)


def _wid():
    return lax.axis_index("s") * NC + lax.axis_index("c")


@functools.partial(
    pl.kernel,
    out_type=jax.ShapeDtypeStruct((NW, N), jnp.float32),
    mesh=_mesh,
    compiler_params=pltpu.CompilerParams(needs_layout_passes=False),
    scratch_types=[
        pltpu.VMEM((EPW,), jnp.int32),
        pltpu.VMEM((EPW,), jnp.float32),
        pltpu.VMEM((N,), jnp.float32),
    ],
)
def _sc_deg_partials(pair_hbm, ew_hbm, degp_hbm, pairv, ewv, degl):
    wid = _wid()
    off = pl.multiple_of(wid * EPW, 8)
    pltpu.sync_copy(pair_hbm.at[pl.ds(off, EPW)], pairv)
    pltpu.sync_copy(ew_hbm.at[pl.ds(off, EPW)], ewv)

    z16 = jnp.zeros((16,), jnp.float32)

    @plsc.parallel_loop(0, N, step=16, unroll=4)
    def _zero(i):
        degl[pl.ds(i, 16)] = z16

    @plsc.parallel_loop(0, EPW, step=16, unroll=4)
    def _edges(i):
        d16 = lax.shift_right_logical(pairv[pl.ds(i, 16)], 16)
        w16 = ewv[pl.ds(i, 16)]
        plsc.addupdate_scatter(degl, [d16], w16)

    pltpu.sync_copy(degl, degp_hbm.at[wid])


@functools.partial(
    pl.kernel,
    out_type=jax.ShapeDtypeStruct((D * N,), jnp.float32),
    mesh=_mesh,
    compiler_params=pltpu.CompilerParams(needs_layout_passes=False),
    scratch_types=[
        pltpu.VMEM((RPW * N,), jnp.float32),
        pltpu.VMEM((RPW * N,), jnp.float32),
        pltpu.VMEM((2, CHUNK), jnp.int32),
        pltpu.VMEM((2, CHUNK), jnp.float32),
        pltpu.SemaphoreType.DMA,
        pltpu.SemaphoreType.DMA,
        pltpu.SemaphoreType.DMA,
        pltpu.SemaphoreType.DMA,
    ],
)
def _sc_edge_agg(gt_hbm, pair_hbm, ew_hbm, st_hbm,
                 gcols, scols, pairv, ewv,
                 sem_p0, sem_p1, sem_w0, sem_w1):
    wid = _wid()
    base = pl.multiple_of(wid * (RPW * N), 8)
    sems = ((sem_p0, sem_w0), (sem_p1, sem_w1))

    def start(b, ci):
        off = pl.multiple_of(ci * CHUNK, 8)
        pltpu.async_copy(pair_hbm.at[pl.ds(off, CHUNK)], pairv.at[b], sems[b][0])
        pltpu.async_copy(ew_hbm.at[pl.ds(off, CHUNK)], ewv.at[b], sems[b][1])

    def wait(b):
        pltpu.make_async_copy(pair_hbm.at[pl.ds(0, CHUNK)], pairv.at[b], sems[b][0]).wait()
        pltpu.make_async_copy(ew_hbm.at[pl.ds(0, CHUNK)], ewv.at[b], sems[b][1]).wait()

    start(0, 0)
    pltpu.sync_copy(gt_hbm.at[pl.ds(base, RPW * N)], gcols)

    z16 = jnp.zeros((16,), jnp.float32)

    @plsc.parallel_loop(0, RPW * N, step=16, unroll=4)
    def _zero(i):
        scols[pl.ds(i, 16)] = z16

    coff = [jnp.full((16,), c * N, jnp.int32) for c in range(RPW)]

    mask16 = jnp.full((16,), 0xFFFF, jnp.int32)

    def process(b):
        @plsc.parallel_loop(0, CHUNK, step=16, unroll=8)
        def _edges(i):
            p16 = pairv[b, pl.ds(i, 16)]
            w16 = ewv[b, pl.ds(i, 16)]
            s16 = p16 & mask16
            d16 = lax.shift_right_logical(p16, 16)
            for c in range(RPW):
                v = plsc.load_gather(gcols, [s16 + coff[c]])
                plsc.addupdate_scatter(scols, [d16 + coff[c]], v * w16)

    def pair_body(p, _):
        c0 = p * 2
        start(1, c0 + 1)
        wait(0)
        process(0)

        @pl.when(p < NCH // 2 - 1)
        def _():
            start(0, c0 + 2)

        wait(1)
        process(1)
        return 0

    lax.fori_loop(0, NCH // 2, pair_body, 0)
    pltpu.sync_copy(scols, st_hbm.at[pl.ds(base, RPW * N)])


def _tc1_body(x_ref, w1_ref, degp_ref, gt_ref, dinv_ref):
    deg = 1.0 + jnp.sum(degp_ref[...], axis=0, keepdims=True)
    di = lax.rsqrt(deg)
    dinv_ref[...] = di
    hxT = lax.dot_general(
        w1_ref[...], x_ref[...], (((0,), (1,)), ((), ())),
        preferred_element_type=jnp.float32,
    )
    gt_ref[...] = hxT * di


_tc1 = pl.pallas_call(
    _tc1_body,
    out_shape=[
        jax.ShapeDtypeStruct((D, N), jnp.float32),
        jax.ShapeDtypeStruct((1, N), jnp.float32),
    ],
)


def _tc2_body(st_ref, gt_ref, dinv_ref, b_ref, w2_ref, g2t_ref):
    di = dinv_ref[...]
    h1 = jnp.maximum((st_ref[...] + gt_ref[...]) * di + b_ref[...], 0.0)
    g2 = lax.dot_general(
        w2_ref[...], h1, (((0,), (0,)), ((), ())),
        preferred_element_type=jnp.float32,
    )
    g2t_ref[...] = g2 * di


_tc2 = pl.pallas_call(
    _tc2_body,
    out_shape=jax.ShapeDtypeStruct((D, N), jnp.float32),
)


def _tc3_body(st_ref, gt_ref, dinv_ref, b_ref, gam_ref, bet_ref, out_ref):
    di = dinv_ref[...]
    h2 = jnp.maximum((st_ref[...] + gt_ref[...]) * di + b_ref[...], 0.0)
    mu = jnp.mean(h2, axis=0, keepdims=True)
    cen = h2 - mu
    var = jnp.mean(cen * cen, axis=0, keepdims=True)
    y = cen * lax.rsqrt(var + 1e-5) * gam_ref[...] + bet_ref[...]
    out_ref[...] = y.T


_tc3 = pl.pallas_call(
    _tc3_body,
    out_shape=jax.ShapeDtypeStruct((N, D), jnp.float32),
)


def kernel(x, edge_index, edge_attr, W1, b1, W2, b2, gamma, beta):
    pair = edge_index[0] | (edge_index[1] << 16)
    b1c = b1.reshape(D, 1)
    b2c = b2.reshape(D, 1)
    gam = gamma.reshape(D, 1)
    bet = beta.reshape(D, 1)

    degp = _sc_deg_partials(pair, edge_attr)
    g1t, dinv = _tc1(x, W1, degp)
    s1t = _sc_edge_agg(g1t.reshape(D * N), pair, edge_attr).reshape(D, N)
    g2t = _tc2(s1t, g1t, dinv, b1c, W2)
    s2t = _sc_edge_agg(g2t.reshape(D * N), pair, edge_attr).reshape(D, N)
    return _tc3(s2t, g2t, dinv, b2c, gam, bet)


# speedup vs baseline: 1.1912x; 1.1912x over previous
import functools

import jax
import jax.numpy as jnp
from jax import lax
from jax.experimental import pallas as pl
from jax.experimental.pallas import tpu as pltpu
from jax.experimental.pallas import tpu_sc as plsc

N, E, D = 10000, 320000, 128
NC, NS = 2, 16
NW = NC * NS
EPW = E // NW
RPW = D // NW
CHUNK = 6400
NCH = E // CHUNK
NB = 1000

_mesh = plsc.VectorSubcoreMesh(
    core_axis_name="c", subcore_axis_name="s", num_cores=NC, num_subcores=NS
)


def _wid():
    return lax.axis_index("s") * NC + lax.axis_index("c")


@functools.partial(
    pl.kernel,
    out_type=jax.ShapeDtypeStruct((NW, N), jnp.float32),
    mesh=_mesh,
    compiler_params=pltpu.CompilerParams(needs_layout_passes=False),
    scratch_types=[
        pltpu.VMEM((EPW,), jnp.int32),
        pltpu.VMEM((EPW,), jnp.float32),
        pltpu.VMEM((N,), jnp.float32),
    ],
)
def _sc_deg_partials(pair_hbm, ew_hbm, degp_hbm, pairv, ewv, degl):
    wid = _wid()
    off = pl.multiple_of(wid * EPW, 8)
    pltpu.sync_copy(pair_hbm.at[pl.ds(off, EPW)], pairv)
    pltpu.sync_copy(ew_hbm.at[pl.ds(off, EPW)], ewv)

    z16 = jnp.zeros((16,), jnp.float32)

    @plsc.parallel_loop(0, N, step=16, unroll=4)
    def _zero(i):
        degl[pl.ds(i, 16)] = z16

    @plsc.parallel_loop(0, EPW, step=16, unroll=4)
    def _edges(i):
        d16 = lax.shift_right_logical(pairv[pl.ds(i, 16)], 16)
        w16 = ewv[pl.ds(i, 16)]
        plsc.addupdate_scatter(degl, [d16], w16)

    pltpu.sync_copy(degl, degp_hbm.at[wid])


@functools.partial(
    pl.kernel,
    out_type=jax.ShapeDtypeStruct((D * N,), jnp.float32),
    mesh=_mesh,
    compiler_params=pltpu.CompilerParams(needs_layout_passes=False),
    scratch_types=[
        pltpu.VMEM(((RPW // 2) * N,), jnp.int32),
        pltpu.VMEM((RPW * N,), jnp.float32),
        pltpu.VMEM((2, CHUNK), jnp.int32),
        pltpu.VMEM((2, CHUNK), jnp.float32),
        pltpu.SemaphoreType.DMA,
        pltpu.SemaphoreType.DMA,
        pltpu.SemaphoreType.DMA,
        pltpu.SemaphoreType.DMA,
    ],
)
def _sc_edge_agg(gp_hbm, pair_hbm, ew_hbm, st_hbm,
                 gpk, scols, pairv, ewv,
                 sem_p0, sem_p1, sem_w0, sem_w1):
    wid = _wid()
    pbase = pl.multiple_of(wid * ((RPW // 2) * N), 8)
    base = pl.multiple_of(wid * (RPW * N), 8)
    sems = ((sem_p0, sem_w0), (sem_p1, sem_w1))

    def start(b, ci):
        off = pl.multiple_of(ci * CHUNK, 8)
        pltpu.async_copy(pair_hbm.at[pl.ds(off, CHUNK)], pairv.at[b], sems[b][0])
        pltpu.async_copy(ew_hbm.at[pl.ds(off, CHUNK)], ewv.at[b], sems[b][1])

    def wait(b):
        pltpu.make_async_copy(pair_hbm.at[pl.ds(0, CHUNK)], pairv.at[b], sems[b][0]).wait()
        pltpu.make_async_copy(ew_hbm.at[pl.ds(0, CHUNK)], ewv.at[b], sems[b][1]).wait()

    start(0, 0)
    pltpu.sync_copy(gp_hbm.at[pl.ds(pbase, (RPW // 2) * N)], gpk)

    z16 = jnp.zeros((16,), jnp.float32)

    @plsc.parallel_loop(0, RPW * N, step=16, unroll=4)
    def _zero(i):
        scols[pl.ds(i, 16)] = z16

    coff = [jnp.full((16,), c * N, jnp.int32) for c in range(RPW)]

    mask16 = jnp.full((16,), 0xFFFF, jnp.int32)
    poff = [jnp.full((16,), c2 * N, jnp.int32) for c2 in range(RPW // 2)]

    def process(b):
        @plsc.parallel_loop(0, CHUNK, step=16, unroll=4)
        def _edges(i):
            p16 = pairv[b, pl.ds(i, 16)]
            w16 = ewv[b, pl.ds(i, 16)]
            s16 = p16 & mask16
            d16 = lax.shift_right_logical(p16, 16)
            for c2 in range(RPW // 2):
                vp = plsc.load_gather(gpk, [s16 + poff[c2]])
                b32 = plsc.bitcast(vp, jnp.bfloat16)
                lo, hi = plsc.unpack(
                    b32, format=plsc.PackFormat.INTERLEAVED,
                    preferred_element_type=jnp.float32)
                plsc.addupdate_scatter(scols, [d16 + coff[2 * c2]], lo * w16)
                plsc.addupdate_scatter(scols, [d16 + coff[2 * c2 + 1]], hi * w16)

    def pair_body(p, _):
        c0 = p * 2
        start(1, c0 + 1)
        wait(0)
        process(0)

        @pl.when(p < NCH // 2 - 1)
        def _():
            start(0, c0 + 2)

        wait(1)
        process(1)
        return 0

    lax.fori_loop(0, NCH // 2, pair_body, 0)
    pltpu.sync_copy(scols, st_hbm.at[pl.ds(base, RPW * N)])


def _pack_bf16_pairs(g):
    gr = g.reshape(D // 2, 2, N)
    pe = lax.bitcast_convert_type(
        gr[:, 0, :].astype(jnp.bfloat16), jnp.uint16).astype(jnp.int32)
    po = lax.bitcast_convert_type(
        gr[:, 1, :].astype(jnp.bfloat16), jnp.uint16).astype(jnp.int32)
    return pe | (po << 16)


def _tc1_body(x_ref, w1_ref, degp_ref, gt_ref, dinv_ref, gp_ref):
    deg = 1.0 + jnp.sum(degp_ref[...], axis=0, keepdims=True)
    di = lax.rsqrt(deg)
    dinv_ref[...] = di
    hxT = lax.dot_general(
        w1_ref[...], x_ref[...], (((0,), (1,)), ((), ())),
        preferred_element_type=jnp.float32,
    )
    g = hxT * di
    gt_ref[...] = g
    gp_ref[...] = _pack_bf16_pairs(g)


_tc1 = pl.pallas_call(
    _tc1_body,
    out_shape=[
        jax.ShapeDtypeStruct((D, N), jnp.float32),
        jax.ShapeDtypeStruct((1, N), jnp.float32),
        jax.ShapeDtypeStruct((D // 2, N), jnp.int32),
    ],
)


def _tc2_body(st_ref, gt_ref, dinv_ref, b_ref, w2_ref, g2t_ref, g2p_ref):
    di = dinv_ref[...]
    h1 = jnp.maximum((st_ref[...] + gt_ref[...]) * di + b_ref[...], 0.0)
    g2 = lax.dot_general(
        w2_ref[...], h1, (((0,), (0,)), ((), ())),
        preferred_element_type=jnp.float32,
    )
    g2 = g2 * di
    g2t_ref[...] = g2
    g2p_ref[...] = _pack_bf16_pairs(g2)


_tc2 = pl.pallas_call(
    _tc2_body,
    out_shape=[
        jax.ShapeDtypeStruct((D, N), jnp.float32),
        jax.ShapeDtypeStruct((D // 2, N), jnp.int32),
    ],
)


def _tc3_body(st_ref, gt_ref, dinv_ref, b_ref, gam_ref, bet_ref, out_ref):
    di = dinv_ref[...]
    h2 = jnp.maximum((st_ref[...] + gt_ref[...]) * di + b_ref[...], 0.0)
    mu = jnp.mean(h2, axis=0, keepdims=True)
    cen = h2 - mu
    var = jnp.mean(cen * cen, axis=0, keepdims=True)
    y = cen * lax.rsqrt(var + 1e-5) * gam_ref[...] + bet_ref[...]
    out_ref[...] = y.T


_tc3 = pl.pallas_call(
    _tc3_body,
    out_shape=jax.ShapeDtypeStruct((N, D), jnp.float32),
)


def kernel(x, edge_index, edge_attr, W1, b1, W2, b2, gamma, beta):
    pair = edge_index[0] | (edge_index[1] << 16)
    b1c = b1.reshape(D, 1)
    b2c = b2.reshape(D, 1)
    gam = gamma.reshape(D, 1)
    bet = beta.reshape(D, 1)

    degp = _sc_deg_partials(pair, edge_attr)
    g1t, dinv, g1p = _tc1(x, W1, degp)
    s1t = _sc_edge_agg(g1p.reshape((D // 2) * N), pair, edge_attr).reshape(D, N)
    g2t, g2p = _tc2(s1t, g1t, dinv, b1c, W2)
    s2t = _sc_edge_agg(g2p.reshape((D // 2) * N), pair, edge_attr).reshape(D, N)
    return _tc3(s2t, g2t, dinv, b2c, gam, bet)


# speedup vs baseline: 1.1913x; 1.0001x over previous
import functools

import jax
import jax.numpy as jnp
from jax import lax
from jax.experimental import pallas as pl
from jax.experimental.pallas import tpu as pltpu
from jax.experimental.pallas import tpu_sc as plsc

N, E, D = 10000, 320000, 128
NC, NS = 2, 16
NW = NC * NS
EPW = E // NW
RPW = D // NW
CHUNK = 6400
NCH = E // CHUNK
NB = 1000

_mesh = plsc.VectorSubcoreMesh(
    core_axis_name="c", subcore_axis_name="s", num_cores=NC, num_subcores=NS
)


def _wid():
    return lax.axis_index("s") * NC + lax.axis_index("c")


@functools.partial(
    pl.kernel,
    out_type=jax.ShapeDtypeStruct((NW, N), jnp.float32),
    mesh=_mesh,
    compiler_params=pltpu.CompilerParams(needs_layout_passes=False),
    scratch_types=[
        pltpu.VMEM((EPW,), jnp.int32),
        pltpu.VMEM((EPW,), jnp.float32),
        pltpu.VMEM((N,), jnp.float32),
    ],
)
def _sc_deg_partials(pair_hbm, ew_hbm, degp_hbm, pairv, ewv, degl):
    wid = _wid()
    off = pl.multiple_of(wid * EPW, 8)
    pltpu.sync_copy(pair_hbm.at[pl.ds(off, EPW)], pairv)
    pltpu.sync_copy(ew_hbm.at[pl.ds(off, EPW)], ewv)

    z16 = jnp.zeros((16,), jnp.float32)

    @plsc.parallel_loop(0, N, step=16, unroll=4)
    def _zero(i):
        degl[pl.ds(i, 16)] = z16

    @plsc.parallel_loop(0, EPW, step=16, unroll=4)
    def _edges(i):
        d16 = lax.shift_right_logical(pairv[pl.ds(i, 16)], 16)
        w16 = ewv[pl.ds(i, 16)]
        plsc.addupdate_scatter(degl, [d16], w16)

    pltpu.sync_copy(degl, degp_hbm.at[wid])


@functools.partial(
    pl.kernel,
    out_type=jax.ShapeDtypeStruct((D * N,), jnp.float32),
    mesh=_mesh,
    compiler_params=pltpu.CompilerParams(needs_layout_passes=False),
    scratch_types=[
        pltpu.VMEM((N,), jnp.int32),
        pltpu.VMEM((N,), jnp.int32),
        pltpu.VMEM((N,), jnp.float32),
        pltpu.VMEM((N,), jnp.float32),
        pltpu.VMEM((N,), jnp.float32),
        pltpu.VMEM((N,), jnp.float32),
        pltpu.VMEM((2, CHUNK), jnp.int32),
        pltpu.VMEM((2, CHUNK), jnp.float32),
        pltpu.SemaphoreType.DMA,
        pltpu.SemaphoreType.DMA,
        pltpu.SemaphoreType.DMA,
        pltpu.SemaphoreType.DMA,
    ],
)
def _sc_edge_agg(gp_hbm, pair_hbm, ew_hbm, st_hbm,
                 gpk0, gpk1, sc0, sc1, sc2, sc3, pairv, ewv,
                 sem_p0, sem_p1, sem_w0, sem_w1):
    wid = _wid()
    pbase = pl.multiple_of(wid * ((RPW // 2) * N), 8)
    base = pl.multiple_of(wid * (RPW * N), 8)
    sems = ((sem_p0, sem_w0), (sem_p1, sem_w1))

    def start(b, ci):
        off = pl.multiple_of(ci * CHUNK, 8)
        pltpu.async_copy(pair_hbm.at[pl.ds(off, CHUNK)], pairv.at[b], sems[b][0])
        pltpu.async_copy(ew_hbm.at[pl.ds(off, CHUNK)], ewv.at[b], sems[b][1])

    def wait(b):
        pltpu.make_async_copy(pair_hbm.at[pl.ds(0, CHUNK)], pairv.at[b], sems[b][0]).wait()
        pltpu.make_async_copy(ew_hbm.at[pl.ds(0, CHUNK)], ewv.at[b], sems[b][1]).wait()

    gpks = (gpk0, gpk1)
    scs = (sc0, sc1, sc2, sc3)

    start(0, 0)
    pltpu.sync_copy(gp_hbm.at[pl.ds(pbase, N)], gpk0)
    pltpu.sync_copy(gp_hbm.at[pl.ds(pbase + N, N)], gpk1)

    z16 = jnp.zeros((16,), jnp.float32)

    @plsc.parallel_loop(0, N, step=16, unroll=4)
    def _zero(i):
        for c in range(RPW):
            scs[c][pl.ds(i, 16)] = z16

    mask16 = jnp.full((16,), 0xFFFF, jnp.int32)

    def process(b):
        @plsc.parallel_loop(0, CHUNK, step=16, unroll=4)
        def _edges(i):
            p16 = pairv[b, pl.ds(i, 16)]
            w16 = ewv[b, pl.ds(i, 16)]
            s16 = p16 & mask16
            d16 = lax.shift_right_logical(p16, 16)
            for c2 in range(RPW // 2):
                vp = plsc.load_gather(gpks[c2], [s16])
                b32 = plsc.bitcast(vp, jnp.bfloat16)
                lo, hi = plsc.unpack(
                    b32, format=plsc.PackFormat.INTERLEAVED,
                    preferred_element_type=jnp.float32)
                plsc.addupdate_scatter(scs[2 * c2], [d16], lo * w16)
                plsc.addupdate_scatter(scs[2 * c2 + 1], [d16], hi * w16)

    def pair_body(p, _):
        c0 = p * 2
        start(1, c0 + 1)
        wait(0)
        process(0)

        @pl.when(p < NCH // 2 - 1)
        def _():
            start(0, c0 + 2)

        wait(1)
        process(1)
        return 0

    lax.fori_loop(0, NCH // 2, pair_body, 0)
    for c in range(RPW):
        pltpu.sync_copy(scs[c], st_hbm.at[pl.ds(base + c * N, N)])


def _pack_bf16_pairs(g):
    gr = g.reshape(D // 2, 2, N)
    pe = lax.bitcast_convert_type(
        gr[:, 0, :].astype(jnp.bfloat16), jnp.uint16).astype(jnp.int32)
    po = lax.bitcast_convert_type(
        gr[:, 1, :].astype(jnp.bfloat16), jnp.uint16).astype(jnp.int32)
    return pe | (po << 16)


def _tc1_body(x_ref, w1_ref, degp_ref, gt_ref, dinv_ref, gp_ref):
    deg = 1.0 + jnp.sum(degp_ref[...], axis=0, keepdims=True)
    di = lax.rsqrt(deg)
    dinv_ref[...] = di
    hxT = lax.dot_general(
        w1_ref[...], x_ref[...], (((0,), (1,)), ((), ())),
        preferred_element_type=jnp.float32,
    )
    g = hxT * di
    gt_ref[...] = g
    gp_ref[...] = _pack_bf16_pairs(g)


_tc1 = pl.pallas_call(
    _tc1_body,
    out_shape=[
        jax.ShapeDtypeStruct((D, N), jnp.float32),
        jax.ShapeDtypeStruct((1, N), jnp.float32),
        jax.ShapeDtypeStruct((D // 2, N), jnp.int32),
    ],
)


def _tc2_body(st_ref, gt_ref, dinv_ref, b_ref, w2_ref, g2t_ref, g2p_ref):
    di = dinv_ref[...]
    h1 = jnp.maximum((st_ref[...] + gt_ref[...]) * di + b_ref[...], 0.0)
    g2 = lax.dot_general(
        w2_ref[...], h1, (((0,), (0,)), ((), ())),
        preferred_element_type=jnp.float32,
    )
    g2 = g2 * di
    g2t_ref[...] = g2
    g2p_ref[...] = _pack_bf16_pairs(g2)


_tc2 = pl.pallas_call(
    _tc2_body,
    out_shape=[
        jax.ShapeDtypeStruct((D, N), jnp.float32),
        jax.ShapeDtypeStruct((D // 2, N), jnp.int32),
    ],
)


def _tc3_body(st_ref, gt_ref, dinv_ref, b_ref, gam_ref, bet_ref, out_ref):
    di = dinv_ref[...]
    h2 = jnp.maximum((st_ref[...] + gt_ref[...]) * di + b_ref[...], 0.0)
    mu = jnp.mean(h2, axis=0, keepdims=True)
    cen = h2 - mu
    var = jnp.mean(cen * cen, axis=0, keepdims=True)
    y = cen * lax.rsqrt(var + 1e-5) * gam_ref[...] + bet_ref[...]
    out_ref[...] = y.T


_tc3 = pl.pallas_call(
    _tc3_body,
    out_shape=jax.ShapeDtypeStruct((N, D), jnp.float32),
)


def kernel(x, edge_index, edge_attr, W1, b1, W2, b2, gamma, beta):
    pair = edge_index[0] | (edge_index[1] << 16)
    b1c = b1.reshape(D, 1)
    b2c = b2.reshape(D, 1)
    gam = gamma.reshape(D, 1)
    bet = beta.reshape(D, 1)

    degp = _sc_deg_partials(pair, edge_attr)
    g1t, dinv, g1p = _tc1(x, W1, degp)
    s1t = _sc_edge_agg(g1p.reshape((D // 2) * N), pair, edge_attr).reshape(D, N)
    g2t, g2p = _tc2(s1t, g1t, dinv, b1c, W2)
    s2t = _sc_edge_agg(g2p.reshape((D // 2) * N), pair, edge_attr).reshape(D, N)
    return _tc3(s2t, g2t, dinv, b2c, gam, bet)


# speedup vs baseline: 1.2511x; 1.0502x over previous
import functools

import jax
import jax.numpy as jnp
from jax import lax
from jax.experimental import pallas as pl
from jax.experimental.pallas import tpu as pltpu
from jax.experimental.pallas import tpu_sc as plsc

N, E, D = 10000, 320000, 128
NC, NS = 2, 16
NW = NC * NS
EPW = E // NW
RPW = D // NW
CHUNK = 6400
NCH = E // CHUNK
NB = 1000

_mesh = plsc.VectorSubcoreMesh(
    core_axis_name="c", subcore_axis_name="s", num_cores=NC, num_subcores=NS
)


def _wid():
    return lax.axis_index("s") * NC + lax.axis_index("c")


@functools.partial(
    pl.kernel,
    out_type=jax.ShapeDtypeStruct((NW, N), jnp.float32),
    mesh=_mesh,
    compiler_params=pltpu.CompilerParams(needs_layout_passes=False),
    scratch_types=[
        pltpu.VMEM((EPW,), jnp.int32),
        pltpu.VMEM((EPW,), jnp.float32),
        pltpu.VMEM((N,), jnp.float32),
    ],
)
def _sc_deg_partials(pair_hbm, ew_hbm, degp_hbm, pairv, ewv, degl):
    wid = _wid()
    off = pl.multiple_of(wid * EPW, 8)
    pltpu.sync_copy(pair_hbm.at[pl.ds(off, EPW)], pairv)
    pltpu.sync_copy(ew_hbm.at[pl.ds(off, EPW)], ewv)

    z16 = jnp.zeros((16,), jnp.float32)

    @plsc.parallel_loop(0, N, step=16, unroll=4)
    def _zero(i):
        degl[pl.ds(i, 16)] = z16

    @plsc.parallel_loop(0, EPW, step=16, unroll=4)
    def _edges(i):
        d16 = lax.shift_right_logical(pairv[pl.ds(i, 16)], 16)
        w16 = ewv[pl.ds(i, 16)]
        plsc.addupdate_scatter(degl, [d16], w16)

    pltpu.sync_copy(degl, degp_hbm.at[wid])


@functools.partial(
    pl.kernel,
    out_type=jax.ShapeDtypeStruct((D, N), jnp.float32),
    mesh=_mesh,
    compiler_params=pltpu.CompilerParams(needs_layout_passes=False),
    scratch_types=[
        pltpu.VMEM((N,), jnp.int32),
        pltpu.VMEM((N,), jnp.int32),
        pltpu.VMEM((N,), jnp.float32),
        pltpu.VMEM((N,), jnp.float32),
        pltpu.VMEM((N,), jnp.float32),
        pltpu.VMEM((N,), jnp.float32),
        pltpu.VMEM((2, CHUNK), jnp.int32),
        pltpu.VMEM((2, CHUNK), jnp.float32),
        pltpu.SemaphoreType.DMA,
        pltpu.SemaphoreType.DMA,
        pltpu.SemaphoreType.DMA,
        pltpu.SemaphoreType.DMA,
    ],
)
def _sc_edge_agg(gp_hbm, pair_hbm, ew_hbm, st_hbm,
                 gpk0, gpk1, sc0, sc1, sc2, sc3, pairv, ewv,
                 sem_p0, sem_p1, sem_w0, sem_w1):
    wid = _wid()
    sems = ((sem_p0, sem_w0), (sem_p1, sem_w1))

    def start(b, ci):
        off = pl.multiple_of(ci * CHUNK, 8)
        pltpu.async_copy(pair_hbm.at[pl.ds(off, CHUNK)], pairv.at[b], sems[b][0])
        pltpu.async_copy(ew_hbm.at[pl.ds(off, CHUNK)], ewv.at[b], sems[b][1])

    def wait(b):
        pltpu.make_async_copy(pair_hbm.at[pl.ds(0, CHUNK)], pairv.at[b], sems[b][0]).wait()
        pltpu.make_async_copy(ew_hbm.at[pl.ds(0, CHUNK)], ewv.at[b], sems[b][1]).wait()

    gpks = (gpk0, gpk1)
    scs = (sc0, sc1, sc2, sc3)

    start(0, 0)
    pltpu.sync_copy(gp_hbm.at[2 * wid], gpk0)
    pltpu.sync_copy(gp_hbm.at[2 * wid + 1], gpk1)

    z16 = jnp.zeros((16,), jnp.float32)

    @plsc.parallel_loop(0, N, step=16, unroll=4)
    def _zero(i):
        for c in range(RPW):
            scs[c][pl.ds(i, 16)] = z16

    mask16 = jnp.full((16,), 0xFFFF, jnp.int32)

    def process(b):
        @plsc.parallel_loop(0, CHUNK, step=16, unroll=4)
        def _edges(i):
            p16 = pairv[b, pl.ds(i, 16)]
            w16 = ewv[b, pl.ds(i, 16)]
            s16 = p16 & mask16
            d16 = lax.shift_right_logical(p16, 16)
            for c2 in range(RPW // 2):
                vp = plsc.load_gather(gpks[c2], [s16])
                b32 = plsc.bitcast(vp, jnp.bfloat16)
                lo, hi = plsc.unpack(
                    b32, format=plsc.PackFormat.INTERLEAVED,
                    preferred_element_type=jnp.float32)
                plsc.addupdate_scatter(scs[2 * c2], [d16], lo * w16)
                plsc.addupdate_scatter(scs[2 * c2 + 1], [d16], hi * w16)

    def pair_body(p, _):
        c0 = p * 2
        start(1, c0 + 1)
        wait(0)
        process(0)

        @pl.when(p < NCH // 2 - 1)
        def _():
            start(0, c0 + 2)

        wait(1)
        process(1)
        return 0

    lax.fori_loop(0, NCH // 2, pair_body, 0)
    for c in range(RPW):
        pltpu.sync_copy(scs[c], st_hbm.at[RPW * wid + c])


def _pack_bf16_pairs(g):
    gr = g.reshape(D // 2, 2, N)
    pe = lax.bitcast_convert_type(
        gr[:, 0, :].astype(jnp.bfloat16), jnp.uint16).astype(jnp.int32)
    po = lax.bitcast_convert_type(
        gr[:, 1, :].astype(jnp.bfloat16), jnp.uint16).astype(jnp.int32)
    return pe | (po << 16)


def _tc1_body(x_ref, w1_ref, degp_ref, gt_ref, dinv_ref, gp_ref):
    deg = 1.0 + jnp.sum(degp_ref[...], axis=0, keepdims=True)
    di = lax.rsqrt(deg)
    dinv_ref[...] = di
    hxT = lax.dot_general(
        w1_ref[...], x_ref[...], (((0,), (1,)), ((), ())),
        preferred_element_type=jnp.float32,
    )
    g = hxT * di
    gt_ref[...] = g
    gp_ref[...] = _pack_bf16_pairs(g)


_tc1 = pl.pallas_call(
    _tc1_body,
    out_shape=[
        jax.ShapeDtypeStruct((D, N), jnp.float32),
        jax.ShapeDtypeStruct((1, N), jnp.float32),
        jax.ShapeDtypeStruct((D // 2, N), jnp.int32),
    ],
)


def _tc2_body(st_ref, gt_ref, dinv_ref, b_ref, w2_ref, g2t_ref, g2p_ref):
    di = dinv_ref[...]
    h1 = jnp.maximum((st_ref[...] + gt_ref[...]) * di + b_ref[...], 0.0)
    g2 = lax.dot_general(
        w2_ref[...], h1, (((0,), (0,)), ((), ())),
        preferred_element_type=jnp.float32,
    )
    g2 = g2 * di
    g2t_ref[...] = g2
    g2p_ref[...] = _pack_bf16_pairs(g2)


_tc2 = pl.pallas_call(
    _tc2_body,
    out_shape=[
        jax.ShapeDtypeStruct((D, N), jnp.float32),
        jax.ShapeDtypeStruct((D // 2, N), jnp.int32),
    ],
)


def _tc3_body(st_ref, gt_ref, dinv_ref, b_ref, gam_ref, bet_ref, out_ref):
    di = dinv_ref[...]
    h2 = jnp.maximum((st_ref[...] + gt_ref[...]) * di + b_ref[...], 0.0)
    mu = jnp.mean(h2, axis=0, keepdims=True)
    cen = h2 - mu
    var = jnp.mean(cen * cen, axis=0, keepdims=True)
    y = cen * lax.rsqrt(var + 1e-5) * gam_ref[...] + bet_ref[...]
    out_ref[...] = y.T


_tc3 = pl.pallas_call(
    _tc3_body,
    out_shape=jax.ShapeDtypeStruct((N, D), jnp.float32),
)


def kernel(x, edge_index, edge_attr, W1, b1, W2, b2, gamma, beta):
    pair = edge_index[0] | (edge_index[1] << 16)
    b1c = b1.reshape(D, 1)
    b2c = b2.reshape(D, 1)
    gam = gamma.reshape(D, 1)
    bet = beta.reshape(D, 1)

    degp = _sc_deg_partials(pair, edge_attr)
    g1t, dinv, g1p = _tc1(x, W1, degp)
    s1t = _sc_edge_agg(g1p, pair, edge_attr)
    g2t, g2p = _tc2(s1t, g1t, dinv, b1c, W2)
    s2t = _sc_edge_agg(g2p, pair, edge_attr)
    return _tc3(s2t, g2t, dinv, b2c, gam, bet)


# speedup vs baseline: 1.3142x; 1.0504x over previous
import functools

import jax
import jax.numpy as jnp
from jax import lax
from jax.experimental import pallas as pl
from jax.experimental.pallas import tpu as pltpu
from jax.experimental.pallas import tpu_sc as plsc

N, E, D = 10000, 320000, 128
NC, NS = 2, 16
NW = NC * NS
EPW = E // NW
RPW = D // NW
CHUNK = 6400
NCH = E // CHUNK
NB = 1000

_mesh = plsc.VectorSubcoreMesh(
    core_axis_name="c", subcore_axis_name="s", num_cores=NC, num_subcores=NS
)


def _wid():
    return lax.axis_index("s") * NC + lax.axis_index("c")


@functools.partial(
    pl.kernel,
    out_type=jax.ShapeDtypeStruct((NW, N), jnp.float32),
    mesh=_mesh,
    compiler_params=pltpu.CompilerParams(needs_layout_passes=False),
    scratch_types=[
        pltpu.VMEM((EPW,), jnp.int32),
        pltpu.VMEM((EPW,), jnp.float32),
        pltpu.VMEM((N,), jnp.float32),
    ],
)
def _sc_deg_partials(pair_hbm, ew_hbm, degp_hbm, pairv, ewv, degl):
    wid = _wid()
    off = pl.multiple_of(wid * EPW, 8)
    pltpu.sync_copy(pair_hbm.at[pl.ds(off, EPW)], pairv)
    pltpu.sync_copy(ew_hbm.at[pl.ds(off, EPW)], ewv)

    z16 = jnp.zeros((16,), jnp.float32)

    @plsc.parallel_loop(0, N, step=16, unroll=4)
    def _zero(i):
        degl[pl.ds(i, 16)] = z16

    @plsc.parallel_loop(0, EPW, step=16, unroll=4)
    def _edges(i):
        d16 = lax.shift_right_logical(pairv[pl.ds(i, 16)], 16)
        w16 = ewv[pl.ds(i, 16)]
        plsc.addupdate_scatter(degl, [d16], w16)

    pltpu.sync_copy(degl, degp_hbm.at[wid])


@functools.partial(
    pl.kernel,
    out_type=jax.ShapeDtypeStruct((D, N), jnp.float32),
    mesh=_mesh,
    compiler_params=pltpu.CompilerParams(needs_layout_passes=False),
    scratch_types=[
        pltpu.VMEM((N,), jnp.int32),
        pltpu.VMEM((N,), jnp.int32),
        pltpu.VMEM((N,), jnp.float32),
        pltpu.VMEM((N,), jnp.float32),
        pltpu.VMEM((N,), jnp.float32),
        pltpu.VMEM((N,), jnp.float32),
        pltpu.VMEM((2, CHUNK), jnp.int32),
        pltpu.VMEM((2, CHUNK), jnp.float32),
        pltpu.SemaphoreType.DMA,
        pltpu.SemaphoreType.DMA,
        pltpu.SemaphoreType.DMA,
        pltpu.SemaphoreType.DMA,
    ],
)
def _sc_edge_agg(gp_hbm, pair_hbm, ew_hbm, st_hbm,
                 gpk0, gpk1, sc0, sc1, sc2, sc3, pairv, ewv,
                 sem_p0, sem_p1, sem_w0, sem_w1):
    wid = _wid()
    sems = ((sem_p0, sem_w0), (sem_p1, sem_w1))

    def start(b, ci):
        off = pl.multiple_of(ci * CHUNK, 8)
        pltpu.async_copy(pair_hbm.at[pl.ds(off, CHUNK)], pairv.at[b], sems[b][0])
        pltpu.async_copy(ew_hbm.at[pl.ds(off, CHUNK)], ewv.at[b], sems[b][1])

    def wait(b):
        pltpu.make_async_copy(pair_hbm.at[pl.ds(0, CHUNK)], pairv.at[b], sems[b][0]).wait()
        pltpu.make_async_copy(ew_hbm.at[pl.ds(0, CHUNK)], ewv.at[b], sems[b][1]).wait()

    gpks = (gpk0, gpk1)
    scs = (sc0, sc1, sc2, sc3)

    start(0, 0)
    pltpu.sync_copy(gp_hbm.at[2 * wid], gpk0)
    pltpu.sync_copy(gp_hbm.at[2 * wid + 1], gpk1)

    z16 = jnp.zeros((16,), jnp.float32)

    @plsc.parallel_loop(0, N, step=16, unroll=4)
    def _zero(i):
        for c in range(RPW):
            scs[c][pl.ds(i, 16)] = z16

    mask16 = jnp.full((16,), 0xFFFF, jnp.int32)

    def process(b):
        @plsc.parallel_loop(0, CHUNK, step=16, unroll=4)
        def _edges(i):
            p16 = pairv[b, pl.ds(i, 16)]
            w16 = ewv[b, pl.ds(i, 16)]
            s16 = p16 & mask16
            d16 = lax.shift_right_logical(p16, 16)
            for c2 in range(RPW // 2):
                vp = plsc.load_gather(gpks[c2], [s16])
                b32 = plsc.bitcast(vp, jnp.bfloat16)
                lo, hi = plsc.unpack(
                    b32, format=plsc.PackFormat.INTERLEAVED,
                    preferred_element_type=jnp.float32)
                plsc.addupdate_scatter(scs[2 * c2], [d16], lo * w16)
                plsc.addupdate_scatter(scs[2 * c2 + 1], [d16], hi * w16)

    def pair_body(p, _):
        c0 = p * 2
        start(1, c0 + 1)
        wait(0)
        process(0)

        @pl.when(p < NCH // 2 - 1)
        def _():
            start(0, c0 + 2)

        wait(1)
        process(1)
        return 0

    lax.fori_loop(0, NCH // 2, pair_body, 0)
    for c2 in range(RPW // 2):
        pltpu.sync_copy(scs[2 * c2], st_hbm.at[2 * wid + c2])
        pltpu.sync_copy(scs[2 * c2 + 1], st_hbm.at[D // 2 + 2 * wid + c2])


def _pack_bf16(g):
    pe = lax.bitcast_convert_type(
        g[: D // 2].astype(jnp.bfloat16), jnp.uint16).astype(jnp.int32)
    po = lax.bitcast_convert_type(
        g[D // 2:].astype(jnp.bfloat16), jnp.uint16).astype(jnp.int32)
    return pe | (po << 16)


def _unpack_bf16(p):
    lo = lax.bitcast_convert_type(p << 16, jnp.float32)
    hi = lax.bitcast_convert_type(p & jnp.int32(-0x10000), jnp.float32)
    return jnp.concatenate([lo, hi], axis=0)


def _tc1_body(x_ref, w1_ref, degp_ref, dinv_ref, gp_ref):
    deg = 1.0 + jnp.sum(degp_ref[...], axis=0, keepdims=True)
    di = lax.rsqrt(deg)
    dinv_ref[...] = di
    hxT = lax.dot_general(
        w1_ref[...], x_ref[...], (((0,), (1,)), ((), ())),
        preferred_element_type=jnp.float32,
    )
    gp_ref[...] = _pack_bf16(hxT * di)


_tc1 = pl.pallas_call(
    _tc1_body,
    out_shape=[
        jax.ShapeDtypeStruct((1, N), jnp.float32),
        jax.ShapeDtypeStruct((D // 2, N), jnp.int32),
    ],
)


def _tc2_body(st_ref, gp_ref, dinv_ref, b_ref, w2_ref, g2p_ref):
    di = dinv_ref[...]
    g1 = _unpack_bf16(gp_ref[...])
    h1 = jnp.maximum((st_ref[...] + g1) * di + b_ref[...], 0.0)
    g2 = lax.dot_general(
        w2_ref[...], h1, (((0,), (0,)), ((), ())),
        preferred_element_type=jnp.float32,
    )
    g2p_ref[...] = _pack_bf16(g2 * di)


_tc2 = pl.pallas_call(
    _tc2_body,
    out_shape=jax.ShapeDtypeStruct((D // 2, N), jnp.int32),
)


def _tc3_body(st_ref, gp_ref, dinv_ref, b_ref, gam_ref, bet_ref, out_ref):
    di = dinv_ref[...]
    g2 = _unpack_bf16(gp_ref[...])
    h2 = jnp.maximum((st_ref[...] + g2) * di + b_ref[...], 0.0)
    mu = jnp.mean(h2, axis=0, keepdims=True)
    cen = h2 - mu
    var = jnp.mean(cen * cen, axis=0, keepdims=True)
    y = cen * lax.rsqrt(var + 1e-5) * gam_ref[...] + bet_ref[...]
    out_ref[...] = y.T


_tc3 = pl.pallas_call(
    _tc3_body,
    out_shape=jax.ShapeDtypeStruct((N, D), jnp.float32),
)


def kernel(x, edge_index, edge_attr, W1, b1, W2, b2, gamma, beta):
    pair = edge_index[0] | (edge_index[1] << 16)
    b1c = b1.reshape(D, 1)
    b2c = b2.reshape(D, 1)
    gam = gamma.reshape(D, 1)
    bet = beta.reshape(D, 1)

    degp = _sc_deg_partials(pair, edge_attr)
    dinv, g1p = _tc1(x, W1, degp)
    s1t = _sc_edge_agg(g1p, pair, edge_attr)
    g2p = _tc2(s1t, g1p, dinv, b1c, W2)
    s2t = _sc_edge_agg(g2p, pair, edge_attr)
    return _tc3(s2t, g2p, dinv, b2c, gam, bet)
